# baseline (speedup 1.0000x reference)
import jax
import jax.numpy as jnp
from jax import lax
from jax.experimental import pallas as pl
from jax.experimental.pallas import tpu as pltpu
from jax.experimental.pallas import tpu_sc as plsc

N_NODES = 10000
D = 128
E = 320000
NC = 2
NS = 16
NW = NC * NS
EPW = E // NW
C = 80
NCHUNK = EPW // C
NG = C // 16


def _dot_body(h_hbm, ei_hbm, out_hbm,
              a_src, a_dst, u0, v0, u1, v1, ob0, ob1,
              su0, sv0, su1, sv1):
    wid = lax.axis_index("s") * NC + lax.axis_index("c")
    base0 = wid * EPW
    pltpu.sync_copy(ei_hbm.at[0, pl.ds(base0, EPW)], a_src)
    pltpu.sync_copy(ei_hbm.at[1, pl.ds(base0, EPW)], a_dst)

    lane = lax.iota(jnp.int32, 16)
    perm = {s: lane ^ s for s in (8, 4, 2, 1)}
    mask = {s: (lane & s) == 0 for s in (8, 4, 2, 1)}
    BITREV = [0, 8, 4, 12, 2, 10, 6, 14, 1, 9, 5, 13, 3, 11, 7, 15]

    def merge(x, y, s):
        m = mask[s]
        a = jnp.where(m, x, y)
        b = jnp.where(m, y, x)
        return a + b.at[perm[s]].get(mode="promise_in_bounds")

    def fire(i, u, v, su, sv):
        pltpu.async_copy(h_hbm.at[a_src.at[pl.ds(i * C, C)]], u, su)
        pltpu.async_copy(h_hbm.at[a_dst.at[pl.ds(i * C, C)]], v, sv)

    def wait(i, u, v, su, sv):
        pltpu.make_async_copy(h_hbm.at[a_src.at[pl.ds(i * C, C)]], u, su).wait()
        pltpu.make_async_copy(h_hbm.at[a_dst.at[pl.ds(i * C, C)]], v, sv).wait()

    def edge_acc(u_rows, v_rows, e):
        prods = []
        for j in range(4):
            wu = u_rows[e, pl.ds(16 * j, 16)]
            wv = v_rows[e, pl.ds(16 * j, 16)]
            prods.append(plsc.bitcast(wu, jnp.bfloat16)
                         * plsc.bitcast(wv, jnp.bfloat16))
        acc = None
        for j in (0, 2):
            pw = plsc.bitcast(prods[j] + prods[j + 1], jnp.int32)
            pa = plsc.bitcast(lax.shift_left(pw, 16), jnp.float32)
            pb = plsc.bitcast(
                jnp.bitwise_and(pw, jnp.int32(-65536)), jnp.float32)
            t = pa + pb
            acc = t if acc is None else acc + t
        return acc

    def edge_accs(u_rows, v_rows, g):
        return tuple(edge_acc(u_rows, v_rows, g * 16 + BITREV[idx])
                     for idx in range(16))

    def tree(accs):
        stack = []
        for a in accs:
            node = (0, a)
            while stack and stack[-1][0] == node[0]:
                lvl, x = stack.pop()
                node = (lvl + 1, merge(x, node[1], (8, 4, 2, 1)[lvl]))
            stack.append(node)
        return stack[0][1]

    def store_out(ob, ci):
        pltpu.sync_copy(ob, out_hbm.at[pl.ds(base0 + ci * C, C)])

    def process(ci, u_rows, v_rows, ob_this, ob_prev, carry):
        new = edge_accs(u_rows, v_rows, 0)
        ob_prev[pl.ds(C - 16, 16)] = tree(carry)
        store_out(ob_prev, ci - 1)
        carry = new
        for g in range(1, NG):
            new = edge_accs(u_rows, v_rows, g)
            ob_this[pl.ds((g - 1) * 16, 16)] = tree(carry)
            carry = new
        return carry

    fire(0, u0, v0, su0, sv0)
    fire(1, u1, v1, su1, sv1)
    wait(0, u0, v0, su0, sv0)
    carry = edge_accs(u0, v0, 0)
    for g in range(1, NG):
        new = edge_accs(u0, v0, g)
        ob0[pl.ds((g - 1) * 16, 16)] = tree(carry)
        carry = new
    fire(2, u0, v0, su0, sv0)

    def body(j, carry):
        c1 = 2 * j + 1
        wait(c1, u1, v1, su1, sv1)
        carry = process(c1, u1, v1, ob1, ob0, carry)
        fire(c1 + 2, u1, v1, su1, sv1)
        c2 = 2 * j + 2
        wait(c2, u0, v0, su0, sv0)
        carry = process(c2, u0, v0, ob0, ob1, carry)
        fire(c2 + 2, u0, v0, su0, sv0)
        return carry

    carry = lax.fori_loop(0, (NCHUNK - 3) // 2, body, carry)

    wait(NCHUNK - 2, u1, v1, su1, sv1)
    carry = process(NCHUNK - 2, u1, v1, ob1, ob0, carry)
    wait(NCHUNK - 1, u0, v0, su0, sv0)
    carry = process(NCHUNK - 1, u0, v0, ob0, ob1, carry)
    ob0[pl.ds(C - 16, 16)] = tree(carry)
    store_out(ob0, NCHUNK - 1)


def kernel(h, edge_index):
    w = lax.bitcast_convert_type(h, jnp.uint32)
    b = (w + jnp.uint32(0x7FFF) + ((w >> 16) & jnp.uint32(1))) >> 16
    h = lax.bitcast_convert_type(
        b[:, : D // 2] | (b[:, D // 2:] << 16), jnp.int32)
    mesh = plsc.VectorSubcoreMesh(core_axis_name="c", subcore_axis_name="s")
    f = pl.kernel(
        _dot_body,
        out_type=jax.ShapeDtypeStruct((E,), jnp.float32),
        mesh=mesh,
        compiler_params=pltpu.CompilerParams(
            needs_layout_passes=False, use_tc_tiling_on_sc=False),
        scratch_types=[
            pltpu.VMEM((EPW,), jnp.int32),
            pltpu.VMEM((EPW,), jnp.int32),
            pltpu.VMEM((C, D // 2), jnp.int32),
            pltpu.VMEM((C, D // 2), jnp.int32),
            pltpu.VMEM((C, D // 2), jnp.int32),
            pltpu.VMEM((C, D // 2), jnp.int32),
            pltpu.VMEM((C,), jnp.float32),
            pltpu.VMEM((C,), jnp.float32),
            pltpu.SemaphoreType.DMA,
            pltpu.SemaphoreType.DMA,
            pltpu.SemaphoreType.DMA,
            pltpu.SemaphoreType.DMA,
        ],
    )
    return f(h, edge_index)


# speedup vs baseline: 2.2057x; 2.2057x over previous
import jax
import jax.numpy as jnp
from jax import lax
from jax.experimental import pallas as pl
from jax.experimental.pallas import tpu as pltpu
from jax.experimental.pallas import tpu_sc as plsc

N_NODES = 10000
D = 128
E = 320000
NC = 2
NS = 16
NW = NC * NS
EPW = E // NW
C = 80
NCHUNK = EPW // C
NG = C // 16
NR = 2 * NG


def _dot_body(h_hbm, ei_hbm, out_hbm,
              a_src, a_dst, u0, v0, u1, v1, ob,
              su0, sv0, su1, sv1):
    wid = lax.axis_index("s") * NC + lax.axis_index("c")
    base0 = wid * EPW
    pltpu.sync_copy(ei_hbm.at[0, pl.ds(base0, EPW)], a_src)
    pltpu.sync_copy(ei_hbm.at[1, pl.ds(base0, EPW)], a_dst)

    lane = lax.iota(jnp.int32, 16)
    perm = {s: lane ^ s for s in (8, 4, 2, 1)}
    mask = {s: (lane & s) == 0 for s in (8, 4, 2, 1)}
    BITREV = [0, 8, 4, 12, 2, 10, 6, 14, 1, 9, 5, 13, 3, 11, 7, 15]

    def merge(x, y, s):
        m = mask[s]
        a = jnp.where(m, x, y)
        b = jnp.where(m, y, x)
        return a + b.at[perm[s]].get(mode="promise_in_bounds")

    def fire(i, u, v, su, sv):
        pltpu.async_copy(h_hbm.at[a_src.at[pl.ds(i * C, C)]], u, su)
        pltpu.async_copy(h_hbm.at[a_dst.at[pl.ds(i * C, C)]], v, sv)

    def wait(i, u, v, su, sv):
        pltpu.make_async_copy(h_hbm.at[a_src.at[pl.ds(i * C, C)]], u, su).wait()
        pltpu.make_async_copy(h_hbm.at[a_dst.at[pl.ds(i * C, C)]], v, sv).wait()

    def edge_acc(u_rows, v_rows, e):
        prods = []
        for j in range(4):
            wu = u_rows[e, pl.ds(16 * j, 16)]
            wv = v_rows[e, pl.ds(16 * j, 16)]
            prods.append(plsc.bitcast(wu, jnp.bfloat16)
                         * plsc.bitcast(wv, jnp.bfloat16))
        acc = None
        for j in (0, 2):
            pw = plsc.bitcast(prods[j] + prods[j + 1], jnp.int32)
            pa = plsc.bitcast(lax.shift_left(pw, 16), jnp.float32)
            pb = plsc.bitcast(
                jnp.bitwise_and(pw, jnp.int32(-65536)), jnp.float32)
            t = pa + pb
            acc = t if acc is None else acc + t
        return acc

    def edge_accs(u_rows, v_rows, g):
        return tuple(edge_acc(u_rows, v_rows, g * 16 + BITREV[idx])
                     for idx in range(16))

    def tree(accs):
        stack = []
        for a in accs:
            node = (0, a)
            while stack and stack[-1][0] == node[0]:
                lvl, x = stack.pop()
                node = (lvl + 1, merge(x, node[1], (8, 4, 2, 1)[lvl]))
            stack.append(node)
        return stack[0][1]

    def make_group_body(u_rows, v_rows, off):
        def group_body(g, carry):
            new = edge_accs(u_rows, v_rows, g)
            pos = lax.rem(g + (off - 1 + NR), NR) * 16
            ob[pl.ds(pos, 16)] = tree(carry)
            return new
        return group_body

    body_even = make_group_body(u0, v0, 0)
    body_odd = make_group_body(u1, v1, NG)

    def flush_even(ci):
        pltpu.sync_copy(ob.at[pl.ds(0, C)],
                        out_hbm.at[pl.ds(base0 + ci * C, C)])

    def flush_odd(ci):
        pltpu.sync_copy(ob.at[pl.ds(C, C)],
                        out_hbm.at[pl.ds(base0 + ci * C, C)])

    fire(0, u0, v0, su0, sv0)
    fire(1, u1, v1, su1, sv1)
    wait(0, u0, v0, su0, sv0)
    carry = edge_accs(u0, v0, 0)
    carry = lax.fori_loop(1, NG, body_even, carry)
    fire(2, u0, v0, su0, sv0)

    def body(j, carry):
        c1 = 2 * j + 1
        wait(c1, u1, v1, su1, sv1)
        carry = lax.fori_loop(0, NG, body_odd, carry)
        flush_even(c1 - 1)
        fire(c1 + 2, u1, v1, su1, sv1)
        c2 = 2 * j + 2
        wait(c2, u0, v0, su0, sv0)
        carry = lax.fori_loop(0, NG, body_even, carry)
        flush_odd(c1)
        fire(c2 + 2, u0, v0, su0, sv0)
        return carry

    carry = lax.fori_loop(0, (NCHUNK - 3) // 2, body, carry)

    wait(NCHUNK - 2, u1, v1, su1, sv1)
    carry = lax.fori_loop(0, NG, body_odd, carry)
    flush_even(NCHUNK - 3)
    wait(NCHUNK - 1, u0, v0, su0, sv0)
    carry = lax.fori_loop(0, NG, body_even, carry)
    flush_odd(NCHUNK - 2)
    ob[pl.ds(C - 16, 16)] = tree(carry)
    flush_even(NCHUNK - 1)


def kernel(h, edge_index):
    w = lax.bitcast_convert_type(h, jnp.uint32)
    b = (w + jnp.uint32(0x7FFF) + ((w >> 16) & jnp.uint32(1))) >> 16
    h = lax.bitcast_convert_type(
        b[:, : D // 2] | (b[:, D // 2:] << 16), jnp.int32)
    mesh = plsc.VectorSubcoreMesh(core_axis_name="c", subcore_axis_name="s")
    f = pl.kernel(
        _dot_body,
        out_type=jax.ShapeDtypeStruct((E,), jnp.float32),
        mesh=mesh,
        compiler_params=pltpu.CompilerParams(
            needs_layout_passes=False, use_tc_tiling_on_sc=False),
        scratch_types=[
            pltpu.VMEM((EPW,), jnp.int32),
            pltpu.VMEM((EPW,), jnp.int32),
            pltpu.VMEM((C, D // 2), jnp.int32),
            pltpu.VMEM((C, D // 2), jnp.int32),
            pltpu.VMEM((C, D // 2), jnp.int32),
            pltpu.VMEM((C, D // 2), jnp.int32),
            pltpu.VMEM((2 * C,), jnp.float32),
            pltpu.SemaphoreType.DMA,
            pltpu.SemaphoreType.DMA,
            pltpu.SemaphoreType.DMA,
            pltpu.SemaphoreType.DMA,
        ],
    )
    return f(h, edge_index)
